# Initial kernel scaffold; baseline (speedup 1.0000x reference)
import jax
import jax.numpy as jnp
from jax.experimental import pallas as pl


def kernel(hidden_states, W):
    raise NotImplementedError("write your pallas kernel here")


# speedup vs baseline: 3.8266x; 3.8266x over previous
import jax
import jax.numpy as jnp
from jax.experimental import pallas as pl
from jax.experimental.pallas import tpu as pltpu

E = 64
H = 768
T = 4 * 8192
TB = 2048
NB = T // TB
CK = 1024
SINKHORN_ITERS = 30
EPS = 1e-8


def _router_kernel(x_ref, wt_ref, logits_ref, affin_ref, idx_ref, cost_ref):
    i = pl.program_id(0)
    logits = jnp.dot(x_ref[...], wt_ref[...], preferred_element_type=jnp.float32)
    logits_ref[...] = logits
    affin_ref[...] = jax.nn.sigmoid(logits)
    cost_ref[:, pl.ds(i * TB, TB)] = jnp.exp(logits).T

    @pl.when(i == NB - 1)
    def _sinkhorn_and_argmax():
        inv_n = jnp.float32(1.0 / T)
        inv_m = jnp.float32(1.0 / E)

        def body(_, d1):
            def chunk(k, acc):
                blk = cost_ref[:, pl.ds(k * CK, CK)]
                r = jnp.sum(blk * d1, axis=0, keepdims=True)
                d0 = inv_n / (r + EPS)
                return acc + jnp.sum(blk * d0, axis=1, keepdims=True)

            c = jax.lax.fori_loop(0, T // CK, chunk,
                                  jnp.zeros((E, 1), jnp.float32))
            return inv_m / (c + EPS)

        d1 = jax.lax.fori_loop(0, SINKHORN_ITERS, body,
                               jnp.ones((E, 1), jnp.float32))

        def argmax_chunk(k, _):
            vals = cost_ref[:, pl.ds(k * CK, CK)] * d1
            m = jnp.max(vals, axis=0, keepdims=True)
            ids = jax.lax.broadcasted_iota(jnp.int32, (E, CK), 0)
            idx_ref[:, pl.ds(k * CK, CK)] = jnp.min(
                jnp.where(vals == m, ids, E), axis=0, keepdims=True)
            return 0

        jax.lax.fori_loop(0, T // CK, argmax_chunk, 0)


@jax.jit
def kernel(hidden_states, W):
    x = hidden_states.reshape(-1, H)
    wt = W.T
    logits, affin, idx = pl.pallas_call(
        _router_kernel,
        grid=(NB,),
        in_specs=[
            pl.BlockSpec((TB, H), lambda i: (i, 0)),
            pl.BlockSpec((H, E), lambda i: (0, 0)),
        ],
        out_specs=[
            pl.BlockSpec((TB, E), lambda i: (i, 0)),
            pl.BlockSpec((TB, E), lambda i: (i, 0)),
            pl.BlockSpec((1, T), lambda i: (0, 0)),
        ],
        out_shape=[
            jax.ShapeDtypeStruct((T, E), jnp.float32),
            jax.ShapeDtypeStruct((T, E), jnp.float32),
            jax.ShapeDtypeStruct((1, T), jnp.int32),
        ],
        scratch_shapes=[pltpu.VMEM((E, T), jnp.float32)],
    )(x, wt)
    return logits, affin, idx.reshape(T, 1)
